# baseline (speedup 1.0000x reference)
import functools

import jax
import jax.numpy as jnp
from jax import lax
from jax.experimental import pallas as pl
from jax.experimental.pallas import tpu as pltpu
from jax.experimental.pallas import tpu_sc as plsc

N = 10000
M = 32
D = 128
DE = 16
F = 256
NM = N * M

NC = 2
NS = 16
NW = NC * NS
CHUNK = 128
NCHUNKS = NM // CHUNK
NBUF = 4
WITERS = -(-(NM // CHUNK) // (NW * NBUF)) * NBUF
NCHUNKS_PAD = WITERS * NW
NM_PAD = NCHUNKS_PAD * CHUNK

B = 400
BM = B * M
NB = N // B


@functools.lru_cache(maxsize=1)
def _make_sc_gather():
    mesh = plsc.VectorSubcoreMesh(core_axis_name="c", subcore_axis_name="s",
                                  num_cores=NC, num_subcores=NS)

    @functools.partial(
        pl.kernel,
        out_type=jax.ShapeDtypeStruct((NM_PAD, D), jnp.float32),
        mesh=mesh,
        scratch_types=(
            [pltpu.VMEM((CHUNK,), jnp.int32) for _ in range(NBUF)]
            + [pltpu.VMEM((CHUNK, D), jnp.float32) for _ in range(NBUF)]
            + [pltpu.SemaphoreType.DMA for _ in range(2 * NBUF)]
        ),
    )
    def sc_gather(table_hbm, idx_hbm, out_hbm, *bufs):
        idx_v = bufs[:NBUF]
        rows_v = bufs[NBUF:2 * NBUF]
        gsem = bufs[2 * NBUF:3 * NBUF]
        wsem = bufs[3 * NBUF:]
        wid = lax.axis_index("s") * NC + lax.axis_index("c")

        def body(j, _):
            @pl.when(j > 0)
            def _w():
                for s in range(NBUF):
                    kp = wid + ((j - 1) * NBUF + s) * NW
                    pltpu.make_async_copy(
                        rows_v[s], out_hbm.at[pl.ds(kp * CHUNK, CHUNK)],
                        wsem[s]).wait()

            cps = []
            for s in range(NBUF):
                k = wid + (j * NBUF + s) * NW
                pltpu.sync_copy(idx_hbm.at[k], idx_v[s])
                cps.append(pltpu.async_copy(
                    table_hbm.at[idx_v[s]], rows_v[s], gsem[s]))
            for s in range(NBUF):
                k = wid + (j * NBUF + s) * NW
                cps[s].wait()
                pltpu.async_copy(
                    rows_v[s], out_hbm.at[pl.ds(k * CHUNK, CHUNK)], wsem[s])
            return 0

        lax.fori_loop(0, WITERS // NBUF, body, 0)
        for s in range(NBUF):
            kp = wid + ((WITERS // NBUF - 1) * NBUF + s) * NW
            pltpu.make_async_copy(
                rows_v[s], out_hbm.at[pl.ds(kp * CHUNK, CHUNK)], wsem[s]).wait()

    return sc_gather


def _stats_body(atom_ref, g_ref, nbrT_ref, idxf_ref, ws_ref, wn_ref,
                we_ref, b_ref, base_ref, ssum_ref, ssq_ref):
    pid = pl.program_id(0)
    atom = atom_ref[...]
    base = jnp.dot(atom, ws_ref[...], preferred_element_type=jnp.float32)
    base = base + b_ref[...]
    base_ref[...] = base

    maskT = (idxf_ref[...] != 0).astype(jnp.bfloat16)
    nbrT = nbrT_ref[...].astype(jnp.bfloat16) * maskT
    gbf = g_ref[...].astype(jnp.bfloat16)
    x = jnp.dot(gbf, wn_ref[...], preferred_element_type=jnp.float32)
    x = x + lax.dot_general(nbrT, we_ref[...],
                            (((0,), (0,)), ((), ())),
                            preferred_element_type=jnp.float32)
    s1 = jnp.sum(x.reshape(B, M, F), axis=1)

    @pl.when(pid == 0)
    def _init():
        ssum_ref[...] = jnp.zeros_like(ssum_ref)
        ssq_ref[...] = jnp.zeros_like(ssq_ref)

    ssum_ref[...] += (M * jnp.sum(base, axis=0, keepdims=True)
                      + jnp.sum(s1, axis=0, keepdims=True))
    ssq_ref[...] += (M * jnp.sum(base * base, axis=0, keepdims=True)
                     + 2.0 * jnp.sum(base * s1, axis=0, keepdims=True)
                     + jnp.sum(x * x, axis=0, keepdims=True))


def _main_body(base_ref, g_ref, nbrT_ref, idxf_ref, idx_ref, wn_ref,
               we_ref, a_ref, bb_ref, s_ref, tsum_ref, tsq_ref):
    pid = pl.program_id(0)
    maskT = (idxf_ref[...] != 0).astype(jnp.bfloat16)
    nbrT = nbrT_ref[...].astype(jnp.bfloat16) * maskT
    gbf = g_ref[...].astype(jnp.bfloat16)
    x = jnp.dot(gbf, wn_ref[...], preferred_element_type=jnp.float32)
    x = x + lax.dot_general(nbrT, we_ref[...],
                            (((0,), (0,)), ((), ())),
                            preferred_element_type=jnp.float32)
    yb = base_ref[...] * a_ref[...] + bb_ref[...]
    rep = jnp.broadcast_to(yb[:, None, :], (B, M, F)).reshape(BM, F)
    y = rep + x

    p = jax.nn.sigmoid(y[:, :D]) * jax.nn.softplus(y[:, D:])
    psum = jnp.sum(p.reshape(B, M, D), axis=1)
    cnt0 = jnp.sum((idx_ref[...] == 0).astype(jnp.float32), axis=1,
                   keepdims=True)
    corr = jax.nn.sigmoid(yb[:, :D]) * jax.nn.softplus(yb[:, D:])
    s = psum - cnt0 * corr
    s_ref[...] = s

    @pl.when(pid == 0)
    def _init():
        tsum_ref[...] = jnp.zeros_like(tsum_ref)
        tsq_ref[...] = jnp.zeros_like(tsq_ref)

    tsum_ref[...] += jnp.sum(s, axis=0, keepdims=True)
    tsq_ref[...] += jnp.sum(s * s, axis=0, keepdims=True)


def _final_body(atom_ref, s_ref, a2_ref, bb2_ref, out_ref):
    y2 = s_ref[...] * a2_ref[...] + bb2_ref[...]
    out_ref[...] = jax.nn.softplus(atom_ref[...] + y2)


def kernel(atom_in_fea, nbr_fea, nbr_fea_idx, W_fc, b_fc, gamma1, beta1,
           gamma2, beta2):
    idx = nbr_fea_idx.astype(jnp.int32)
    iflat = jnp.concatenate(
        [jnp.where(idx == 0, N, idx).reshape(NM),
         jnp.full((NM_PAD - NM,), N, jnp.int32)]).reshape(NCHUNKS_PAD, CHUNK)
    table = jnp.concatenate(
        [atom_in_fea, jnp.zeros((1, D), jnp.float32)], axis=0)
    nbrT = jnp.transpose(nbr_fea, (2, 0, 1)).reshape(DE, NM)
    idxf = idx.reshape(1, NM)

    Ws = W_fc[:, :D].T
    Wn = W_fc[:, D:2 * D].T
    Wnb = Wn.astype(jnp.bfloat16)
    We = W_fc[:, 2 * D:].T.astype(jnp.bfloat16)
    bvec = b_fc.reshape(1, F)

    g = _make_sc_gather()(table, iflat)

    base, ssum, ssq = pl.pallas_call(
        _stats_body,
        grid=(NB,),
        in_specs=[
            pl.BlockSpec((B, D), lambda b: (b, 0)),
            pl.BlockSpec((BM, D), lambda b: (b, 0)),
            pl.BlockSpec((DE, BM), lambda b: (0, b)),
            pl.BlockSpec((1, BM), lambda b: (0, b)),
            pl.BlockSpec((D, F), lambda b: (0, 0)),
            pl.BlockSpec((D, F), lambda b: (0, 0)),
            pl.BlockSpec((DE, F), lambda b: (0, 0)),
            pl.BlockSpec((1, F), lambda b: (0, 0)),
        ],
        out_specs=[
            pl.BlockSpec((B, F), lambda b: (b, 0)),
            pl.BlockSpec((1, F), lambda b: (0, 0)),
            pl.BlockSpec((1, F), lambda b: (0, 0)),
        ],
        out_shape=[
            jax.ShapeDtypeStruct((N, F), jnp.float32),
            jax.ShapeDtypeStruct((1, F), jnp.float32),
            jax.ShapeDtypeStruct((1, F), jnp.float32),
        ],
    )(atom_in_fea, g, nbrT, idxf, Ws, Wnb, We, bvec)

    mu1 = ssum / NM
    var1 = ssq / NM - mu1 * mu1
    a1 = lax.rsqrt(var1 + 1e-5) * gamma1.reshape(1, F)
    bb1 = beta1.reshape(1, F) - mu1 * a1
    Wn_s = (Wn * a1).astype(jnp.bfloat16)
    We_s = (We.astype(jnp.float32) * a1).astype(jnp.bfloat16)

    s, tsum, tsq = pl.pallas_call(
        _main_body,
        grid=(NB,),
        in_specs=[
            pl.BlockSpec((B, F), lambda b: (b, 0)),
            pl.BlockSpec((BM, D), lambda b: (b, 0)),
            pl.BlockSpec((DE, BM), lambda b: (0, b)),
            pl.BlockSpec((1, BM), lambda b: (0, b)),
            pl.BlockSpec((B, M), lambda b: (b, 0)),
            pl.BlockSpec((D, F), lambda b: (0, 0)),
            pl.BlockSpec((DE, F), lambda b: (0, 0)),
            pl.BlockSpec((1, F), lambda b: (0, 0)),
            pl.BlockSpec((1, F), lambda b: (0, 0)),
        ],
        out_specs=[
            pl.BlockSpec((B, D), lambda b: (b, 0)),
            pl.BlockSpec((1, D), lambda b: (0, 0)),
            pl.BlockSpec((1, D), lambda b: (0, 0)),
        ],
        out_shape=[
            jax.ShapeDtypeStruct((N, D), jnp.float32),
            jax.ShapeDtypeStruct((1, D), jnp.float32),
            jax.ShapeDtypeStruct((1, D), jnp.float32),
        ],
    )(base, g, nbrT, idxf, idx, Wn_s, We_s, a1, bb1)

    mu2 = tsum / N
    var2 = tsq / N - mu2 * mu2
    a2 = lax.rsqrt(var2 + 1e-5) * gamma2.reshape(1, D)
    bb2 = beta2.reshape(1, D) - mu2 * a2

    out = pl.pallas_call(
        _final_body,
        grid=(NB,),
        in_specs=[
            pl.BlockSpec((B, D), lambda b: (b, 0)),
            pl.BlockSpec((B, D), lambda b: (b, 0)),
            pl.BlockSpec((1, D), lambda b: (0, 0)),
            pl.BlockSpec((1, D), lambda b: (0, 0)),
        ],
        out_specs=pl.BlockSpec((B, D), lambda b: (b, 0)),
        out_shape=jax.ShapeDtypeStruct((N, D), jnp.float32),
    )(atom_in_fea, s, a2, bb2)
    return out


# speedup vs baseline: 1.3371x; 1.3371x over previous
import functools

import jax
import jax.numpy as jnp
from jax import lax
from jax.experimental import pallas as pl
from jax.experimental.pallas import tpu as pltpu
from jax.experimental.pallas import tpu_sc as plsc

N = 10000
M = 32
D = 128
DE = 16
F = 256
NM = N * M

NC = 2
NS = 16
NW = NC * NS
CHUNK = 128
NCHUNKS = NM // CHUNK
NM_PAD = NM

B = 400
BM = B * M
NB = N // B


@functools.lru_cache(maxsize=1)
def _make_sc_gather():
    mesh = plsc.VectorSubcoreMesh(core_axis_name="c", subcore_axis_name="s",
                                  num_cores=NC, num_subcores=NS)

    @functools.partial(
        pl.kernel,
        out_type=jax.ShapeDtypeStruct((NM_PAD, D), jnp.float32),
        mesh=mesh,
        scratch_types=[
            pltpu.VMEM((CHUNK,), jnp.int32),
            pltpu.VMEM((CHUNK, D), jnp.float32),
            pltpu.SemaphoreType.DMA,
        ],
    )
    def sc_gather(table_hbm, idx_hbm, out_hbm, idx_v, rows_v, sem):
        wid = lax.axis_index("s") * NC + lax.axis_index("c")
        iters = (NCHUNKS + NW - 1) // NW

        def body(i, _):
            k = wid + i * NW

            @pl.when(k < NCHUNKS)
            def _do():
                pltpu.sync_copy(idx_hbm.at[k], idx_v)
                pltpu.async_copy(table_hbm.at[idx_v], rows_v, sem).wait()
                pltpu.sync_copy(rows_v, out_hbm.at[pl.ds(k * CHUNK, CHUNK)])

            return 0

        lax.fori_loop(0, iters, body, 0)

    return sc_gather


def _stats_body(atom_ref, g_ref, nbrT_ref, idxf_ref, ws_ref, wn_ref,
                we_ref, b_ref, base_ref, ssum_ref, ssq_ref):
    pid = pl.program_id(0)
    atom = atom_ref[...]
    base = jnp.dot(atom, ws_ref[...], preferred_element_type=jnp.float32)
    base = base + b_ref[...]
    base_ref[...] = base

    maskT = (idxf_ref[...] != 0).astype(jnp.bfloat16)
    nbrT = nbrT_ref[...].astype(jnp.bfloat16) * maskT
    gbf = g_ref[...].astype(jnp.bfloat16)
    x = jnp.dot(gbf, wn_ref[...], preferred_element_type=jnp.float32)
    x = x + lax.dot_general(nbrT, we_ref[...],
                            (((0,), (0,)), ((), ())),
                            preferred_element_type=jnp.float32)
    s1 = jnp.sum(x.reshape(B, M, F), axis=1)

    @pl.when(pid == 0)
    def _init():
        ssum_ref[...] = jnp.zeros_like(ssum_ref)
        ssq_ref[...] = jnp.zeros_like(ssq_ref)

    ssum_ref[...] += (M * jnp.sum(base, axis=0, keepdims=True)
                      + jnp.sum(s1, axis=0, keepdims=True))
    ssq_ref[...] += (M * jnp.sum(base * base, axis=0, keepdims=True)
                     + 2.0 * jnp.sum(base * s1, axis=0, keepdims=True)
                     + jnp.sum(x * x, axis=0, keepdims=True))


def _main_body(base_ref, g_ref, nbrT_ref, idxf_ref, idx_ref, wn_ref,
               we_ref, a_ref, bb_ref, s_ref, tsum_ref, tsq_ref):
    pid = pl.program_id(0)
    maskT = (idxf_ref[...] != 0).astype(jnp.bfloat16)
    nbrT = nbrT_ref[...].astype(jnp.bfloat16) * maskT
    gbf = g_ref[...].astype(jnp.bfloat16)
    x = jnp.dot(gbf, wn_ref[...], preferred_element_type=jnp.float32)
    x = x + lax.dot_general(nbrT, we_ref[...],
                            (((0,), (0,)), ((), ())),
                            preferred_element_type=jnp.float32)
    yb = base_ref[...] * a_ref[...] + bb_ref[...]
    rep = jnp.broadcast_to(yb[:, None, :], (B, M, F)).reshape(BM, F)
    y = rep + x

    p = jax.nn.sigmoid(y[:, :D]) * jax.nn.softplus(y[:, D:])
    psum = jnp.sum(p.reshape(B, M, D), axis=1)
    cnt0 = jnp.sum((idx_ref[...] == 0).astype(jnp.float32), axis=1,
                   keepdims=True)
    corr = jax.nn.sigmoid(yb[:, :D]) * jax.nn.softplus(yb[:, D:])
    s = psum - cnt0 * corr
    s_ref[...] = s

    @pl.when(pid == 0)
    def _init():
        tsum_ref[...] = jnp.zeros_like(tsum_ref)
        tsq_ref[...] = jnp.zeros_like(tsq_ref)

    tsum_ref[...] += jnp.sum(s, axis=0, keepdims=True)
    tsq_ref[...] += jnp.sum(s * s, axis=0, keepdims=True)


def _final_body(atom_ref, s_ref, a2_ref, bb2_ref, out_ref):
    y2 = s_ref[...] * a2_ref[...] + bb2_ref[...]
    out_ref[...] = jax.nn.softplus(atom_ref[...] + y2)


def kernel(atom_in_fea, nbr_fea, nbr_fea_idx, W_fc, b_fc, gamma1, beta1,
           gamma2, beta2):
    idx = nbr_fea_idx.astype(jnp.int32)
    iflat = jnp.where(idx == 0, N, idx).reshape(NCHUNKS, CHUNK)
    table = jnp.concatenate(
        [atom_in_fea, jnp.zeros((1, D), jnp.float32)], axis=0)
    nbrT = jnp.transpose(nbr_fea, (2, 0, 1)).reshape(DE, NM)
    idxf = idx.reshape(1, NM)

    Ws = W_fc[:, :D].T
    Wn = W_fc[:, D:2 * D].T
    Wnb = Wn.astype(jnp.bfloat16)
    We = W_fc[:, 2 * D:].T.astype(jnp.bfloat16)
    bvec = b_fc.reshape(1, F)

    g = _make_sc_gather()(table, iflat)

    base, ssum, ssq = pl.pallas_call(
        _stats_body,
        grid=(NB,),
        in_specs=[
            pl.BlockSpec((B, D), lambda b: (b, 0)),
            pl.BlockSpec((BM, D), lambda b: (b, 0)),
            pl.BlockSpec((DE, BM), lambda b: (0, b)),
            pl.BlockSpec((1, BM), lambda b: (0, b)),
            pl.BlockSpec((D, F), lambda b: (0, 0)),
            pl.BlockSpec((D, F), lambda b: (0, 0)),
            pl.BlockSpec((DE, F), lambda b: (0, 0)),
            pl.BlockSpec((1, F), lambda b: (0, 0)),
        ],
        out_specs=[
            pl.BlockSpec((B, F), lambda b: (b, 0)),
            pl.BlockSpec((1, F), lambda b: (0, 0)),
            pl.BlockSpec((1, F), lambda b: (0, 0)),
        ],
        out_shape=[
            jax.ShapeDtypeStruct((N, F), jnp.float32),
            jax.ShapeDtypeStruct((1, F), jnp.float32),
            jax.ShapeDtypeStruct((1, F), jnp.float32),
        ],
    )(atom_in_fea, g, nbrT, idxf, Ws, Wnb, We, bvec)

    mu1 = ssum / NM
    var1 = ssq / NM - mu1 * mu1
    a1 = lax.rsqrt(var1 + 1e-5) * gamma1.reshape(1, F)
    bb1 = beta1.reshape(1, F) - mu1 * a1
    Wn_s = (Wn * a1).astype(jnp.bfloat16)
    We_s = (We.astype(jnp.float32) * a1).astype(jnp.bfloat16)

    s, tsum, tsq = pl.pallas_call(
        _main_body,
        grid=(NB,),
        in_specs=[
            pl.BlockSpec((B, F), lambda b: (b, 0)),
            pl.BlockSpec((BM, D), lambda b: (b, 0)),
            pl.BlockSpec((DE, BM), lambda b: (0, b)),
            pl.BlockSpec((1, BM), lambda b: (0, b)),
            pl.BlockSpec((B, M), lambda b: (b, 0)),
            pl.BlockSpec((D, F), lambda b: (0, 0)),
            pl.BlockSpec((DE, F), lambda b: (0, 0)),
            pl.BlockSpec((1, F), lambda b: (0, 0)),
            pl.BlockSpec((1, F), lambda b: (0, 0)),
        ],
        out_specs=[
            pl.BlockSpec((B, D), lambda b: (b, 0)),
            pl.BlockSpec((1, D), lambda b: (0, 0)),
            pl.BlockSpec((1, D), lambda b: (0, 0)),
        ],
        out_shape=[
            jax.ShapeDtypeStruct((N, D), jnp.float32),
            jax.ShapeDtypeStruct((1, D), jnp.float32),
            jax.ShapeDtypeStruct((1, D), jnp.float32),
        ],
    )(base, g, nbrT, idxf, idx, Wn_s, We_s, a1, bb1)

    mu2 = tsum / N
    var2 = tsq / N - mu2 * mu2
    a2 = lax.rsqrt(var2 + 1e-5) * gamma2.reshape(1, D)
    bb2 = beta2.reshape(1, D) - mu2 * a2

    out = pl.pallas_call(
        _final_body,
        grid=(NB,),
        in_specs=[
            pl.BlockSpec((B, D), lambda b: (b, 0)),
            pl.BlockSpec((B, D), lambda b: (b, 0)),
            pl.BlockSpec((1, D), lambda b: (0, 0)),
            pl.BlockSpec((1, D), lambda b: (0, 0)),
        ],
        out_specs=pl.BlockSpec((B, D), lambda b: (b, 0)),
        out_shape=jax.ShapeDtypeStruct((N, D), jnp.float32),
    )(atom_in_fea, s, a2, bb2)
    return out


# speedup vs baseline: 1.4058x; 1.0514x over previous
import functools

import jax
import jax.numpy as jnp
from jax import lax
from jax.experimental import pallas as pl
from jax.experimental.pallas import tpu as pltpu
from jax.experimental.pallas import tpu_sc as plsc

N = 10000
M = 32
D = 128
DE = 16
F = 256
NM = N * M

NC = 2
NS = 16
NW = NC * NS
CHUNK = 128
HALVES = 2
NH = N // HALVES
NMH = NH * M
NCHUNKS_H = NMH // CHUNK

B = 200
BM = B * M
NBH = NH // B


@functools.lru_cache(maxsize=1)
def _make_sc_gather():
    mesh = plsc.VectorSubcoreMesh(core_axis_name="c", subcore_axis_name="s",
                                  num_cores=NC, num_subcores=NS)

    @functools.partial(
        pl.kernel,
        out_type=jax.ShapeDtypeStruct((NMH, D), jnp.float32),
        mesh=mesh,
        scratch_types=[
            pltpu.VMEM((CHUNK,), jnp.int32),
            pltpu.VMEM((CHUNK, D), jnp.float32),
            pltpu.SemaphoreType.DMA,
        ],
    )
    def sc_gather(table_hbm, idx_hbm, out_hbm, idx_v, rows_v, sem):
        wid = lax.axis_index("s") * NC + lax.axis_index("c")
        iters = (NCHUNKS_H + NW - 1) // NW

        def body(i, _):
            k = wid + i * NW

            @pl.when(k < NCHUNKS_H)
            def _do():
                pltpu.sync_copy(idx_hbm.at[k], idx_v)
                pltpu.async_copy(table_hbm.at[idx_v], rows_v, sem).wait()
                pltpu.sync_copy(rows_v, out_hbm.at[pl.ds(k * CHUNK, CHUNK)])

            return 0

        lax.fori_loop(0, iters, body, 0)

    return sc_gather


def _stats_body(atom_ref, g_ref, nbrT_ref, idxf_ref, ws_ref, wn_ref,
                we_ref, b_ref, base_ref, ssum_ref, ssq_ref):
    pid = pl.program_id(0)
    atom = atom_ref[...]
    base = jnp.dot(atom, ws_ref[...], preferred_element_type=jnp.float32)
    base = base + b_ref[...]
    base_ref[...] = base

    maskT = (idxf_ref[...] != 0).astype(jnp.bfloat16)
    nbrT = nbrT_ref[...].astype(jnp.bfloat16) * maskT
    gbf = g_ref[...].astype(jnp.bfloat16)
    x = jnp.dot(gbf, wn_ref[...], preferred_element_type=jnp.float32)
    x = x + lax.dot_general(nbrT, we_ref[...],
                            (((0,), (0,)), ((), ())),
                            preferred_element_type=jnp.float32)
    s1 = jnp.sum(x.reshape(B, M, F), axis=1)

    @pl.when(pid == 0)
    def _init():
        ssum_ref[...] = jnp.zeros_like(ssum_ref)
        ssq_ref[...] = jnp.zeros_like(ssq_ref)

    ssum_ref[...] += (M * jnp.sum(base, axis=0, keepdims=True)
                      + jnp.sum(s1, axis=0, keepdims=True))
    ssq_ref[...] += (M * jnp.sum(base * base, axis=0, keepdims=True)
                     + 2.0 * jnp.sum(base * s1, axis=0, keepdims=True)
                     + jnp.sum(x * x, axis=0, keepdims=True))


def _main_body(base_ref, g_ref, nbrT_ref, idxf_ref, idx_ref, wn_ref,
               we_ref, a_ref, bb_ref, s_ref, tsum_ref, tsq_ref):
    pid = pl.program_id(0)
    maskT = (idxf_ref[...] != 0).astype(jnp.bfloat16)
    nbrT = nbrT_ref[...].astype(jnp.bfloat16) * maskT
    gbf = g_ref[...].astype(jnp.bfloat16)
    x = jnp.dot(gbf, wn_ref[...], preferred_element_type=jnp.float32)
    x = x + lax.dot_general(nbrT, we_ref[...],
                            (((0,), (0,)), ((), ())),
                            preferred_element_type=jnp.float32)
    yb = base_ref[...] * a_ref[...] + bb_ref[...]
    rep = jnp.broadcast_to(yb[:, None, :], (B, M, F)).reshape(BM, F)
    y = rep + x

    p = jax.nn.sigmoid(y[:, :D]) * jax.nn.softplus(y[:, D:])
    psum = jnp.sum(p.reshape(B, M, D), axis=1)
    cnt0 = jnp.sum((idx_ref[...] == 0).astype(jnp.float32), axis=1,
                   keepdims=True)
    corr = jax.nn.sigmoid(yb[:, :D]) * jax.nn.softplus(yb[:, D:])
    s = psum - cnt0 * corr
    s_ref[...] = s

    @pl.when(pid == 0)
    def _init():
        tsum_ref[...] = jnp.zeros_like(tsum_ref)
        tsq_ref[...] = jnp.zeros_like(tsq_ref)

    tsum_ref[...] += jnp.sum(s, axis=0, keepdims=True)
    tsq_ref[...] += jnp.sum(s * s, axis=0, keepdims=True)


def _final_body(atom_ref, s_ref, a2_ref, bb2_ref, out_ref):
    y2 = s_ref[...] * a2_ref[...] + bb2_ref[...]
    out_ref[...] = jax.nn.softplus(atom_ref[...] + y2)


def _stats_call(h, atom, g_h, nbrT, idxf, Ws, Wnb, We, bvec):
    oa = h * NBH
    return pl.pallas_call(
        _stats_body,
        grid=(NBH,),
        in_specs=[
            pl.BlockSpec((B, D), lambda b: (b + oa, 0)),
            pl.BlockSpec((BM, D), lambda b: (b, 0)),
            pl.BlockSpec((DE, BM), lambda b: (0, b + oa)),
            pl.BlockSpec((1, BM), lambda b: (0, b + oa)),
            pl.BlockSpec((D, F), lambda b: (0, 0)),
            pl.BlockSpec((D, F), lambda b: (0, 0)),
            pl.BlockSpec((DE, F), lambda b: (0, 0)),
            pl.BlockSpec((1, F), lambda b: (0, 0)),
        ],
        out_specs=[
            pl.BlockSpec((B, F), lambda b: (b, 0)),
            pl.BlockSpec((1, F), lambda b: (0, 0)),
            pl.BlockSpec((1, F), lambda b: (0, 0)),
        ],
        out_shape=[
            jax.ShapeDtypeStruct((NH, F), jnp.float32),
            jax.ShapeDtypeStruct((1, F), jnp.float32),
            jax.ShapeDtypeStruct((1, F), jnp.float32),
        ],
    )(atom, g_h, nbrT, idxf, Ws, Wnb, We, bvec)


def _main_call(h, base_h, g_h, nbrT, idxf, idx, Wn_s, We_s, a1, bb1):
    oa = h * NBH
    return pl.pallas_call(
        _main_body,
        grid=(NBH,),
        in_specs=[
            pl.BlockSpec((B, F), lambda b: (b, 0)),
            pl.BlockSpec((BM, D), lambda b: (b, 0)),
            pl.BlockSpec((DE, BM), lambda b: (0, b + oa)),
            pl.BlockSpec((1, BM), lambda b: (0, b + oa)),
            pl.BlockSpec((B, M), lambda b: (b + oa, 0)),
            pl.BlockSpec((D, F), lambda b: (0, 0)),
            pl.BlockSpec((DE, F), lambda b: (0, 0)),
            pl.BlockSpec((1, F), lambda b: (0, 0)),
            pl.BlockSpec((1, F), lambda b: (0, 0)),
        ],
        out_specs=[
            pl.BlockSpec((B, D), lambda b: (b, 0)),
            pl.BlockSpec((1, D), lambda b: (0, 0)),
            pl.BlockSpec((1, D), lambda b: (0, 0)),
        ],
        out_shape=[
            jax.ShapeDtypeStruct((NH, D), jnp.float32),
            jax.ShapeDtypeStruct((1, D), jnp.float32),
            jax.ShapeDtypeStruct((1, D), jnp.float32),
        ],
    )(base_h, g_h, nbrT, idxf, idx, Wn_s, We_s, a1, bb1)


def kernel(atom_in_fea, nbr_fea, nbr_fea_idx, W_fc, b_fc, gamma1, beta1,
           gamma2, beta2):
    idx = nbr_fea_idx.astype(jnp.int32)
    iflat = jnp.where(idx == 0, N, idx).reshape(HALVES, NCHUNKS_H, CHUNK)
    table = jnp.concatenate(
        [atom_in_fea, jnp.zeros((1, D), jnp.float32)], axis=0)
    nbrT = jnp.transpose(nbr_fea, (2, 0, 1)).reshape(DE, NM)
    idxf = idx.reshape(1, NM)

    Ws = W_fc[:, :D].T
    Wn = W_fc[:, D:2 * D].T
    Wnb = Wn.astype(jnp.bfloat16)
    We = W_fc[:, 2 * D:].T.astype(jnp.bfloat16)
    bvec = b_fc.reshape(1, F)

    gather = _make_sc_gather()
    g = [gather(table, iflat[h]) for h in range(HALVES)]

    stats = [
        _stats_call(h, atom_in_fea, g[h], nbrT, idxf, Ws, Wnb, We, bvec)
        for h in range(HALVES)
    ]
    ssum = sum(st[1] for st in stats)
    ssq = sum(st[2] for st in stats)

    mu1 = ssum / NM
    var1 = ssq / NM - mu1 * mu1
    a1 = lax.rsqrt(var1 + 1e-5) * gamma1.reshape(1, F)
    bb1 = beta1.reshape(1, F) - mu1 * a1
    Wn_s = (Wn * a1).astype(jnp.bfloat16)
    We_s = (We.astype(jnp.float32) * a1).astype(jnp.bfloat16)

    mains = [
        _main_call(h, stats[h][0], g[h], nbrT, idxf, idx, Wn_s, We_s, a1, bb1)
        for h in range(HALVES)
    ]
    tsum = sum(mn[1] for mn in mains)
    tsq = sum(mn[2] for mn in mains)

    mu2 = tsum / N
    var2 = tsq / N - mu2 * mu2
    a2 = lax.rsqrt(var2 + 1e-5) * gamma2.reshape(1, D)
    bb2 = beta2.reshape(1, D) - mu2 * a2

    s_all = jnp.concatenate([mn[0] for mn in mains], axis=0)
    out = pl.pallas_call(
        _final_body,
        grid=(HALVES * NBH,),
        in_specs=[
            pl.BlockSpec((B, D), lambda b: (b, 0)),
            pl.BlockSpec((B, D), lambda b: (b, 0)),
            pl.BlockSpec((1, D), lambda b: (0, 0)),
            pl.BlockSpec((1, D), lambda b: (0, 0)),
        ],
        out_specs=pl.BlockSpec((B, D), lambda b: (b, 0)),
        out_shape=jax.ShapeDtypeStruct((N, D), jnp.float32),
    )(atom_in_fea, s_all, a2, bb2)
    return out


# speedup vs baseline: 1.4206x; 1.0105x over previous
import functools

import jax
import jax.numpy as jnp
from jax import lax
from jax.experimental import pallas as pl
from jax.experimental.pallas import tpu as pltpu
from jax.experimental.pallas import tpu_sc as plsc

N = 10000
M = 32
D = 128
DE = 16
F = 256
NM = N * M

NC = 2
NS = 16
NW = NC * NS
CHUNK = 128
HALVES = 5
NH = N // HALVES
NMH = NH * M
NCHUNKS_H = NMH // CHUNK

B = 200
BM = B * M
NBH = NH // B


@functools.lru_cache(maxsize=1)
def _make_sc_gather():
    mesh = plsc.VectorSubcoreMesh(core_axis_name="c", subcore_axis_name="s",
                                  num_cores=NC, num_subcores=NS)

    @functools.partial(
        pl.kernel,
        out_type=jax.ShapeDtypeStruct((NMH, D), jnp.float32),
        mesh=mesh,
        scratch_types=[
            pltpu.VMEM((CHUNK,), jnp.int32),
            pltpu.VMEM((CHUNK, D), jnp.float32),
            pltpu.SemaphoreType.DMA,
        ],
    )
    def sc_gather(table_hbm, idx_hbm, out_hbm, idx_v, rows_v, sem):
        wid = lax.axis_index("s") * NC + lax.axis_index("c")
        iters = (NCHUNKS_H + NW - 1) // NW

        def body(i, _):
            k = wid + i * NW

            @pl.when(k < NCHUNKS_H)
            def _do():
                pltpu.sync_copy(idx_hbm.at[k], idx_v)
                pltpu.async_copy(table_hbm.at[idx_v], rows_v, sem).wait()
                pltpu.sync_copy(rows_v, out_hbm.at[pl.ds(k * CHUNK, CHUNK)])

            return 0

        lax.fori_loop(0, iters, body, 0)

    return sc_gather


def _stats_body(atom_ref, g_ref, nbrT_ref, idxf_ref, ws_ref, wn_ref,
                we_ref, b_ref, base_ref, ssum_ref, ssq_ref):
    pid = pl.program_id(0)
    atom = atom_ref[...]
    base = jnp.dot(atom, ws_ref[...], preferred_element_type=jnp.float32)
    base = base + b_ref[...]
    base_ref[...] = base

    maskT = (idxf_ref[...] != 0).astype(jnp.bfloat16)
    nbrT = nbrT_ref[...].astype(jnp.bfloat16) * maskT
    gbf = g_ref[...].astype(jnp.bfloat16)
    x = jnp.dot(gbf, wn_ref[...], preferred_element_type=jnp.float32)
    x = x + lax.dot_general(nbrT, we_ref[...],
                            (((0,), (0,)), ((), ())),
                            preferred_element_type=jnp.float32)
    s1 = jnp.sum(x.reshape(B, M, F), axis=1)

    @pl.when(pid == 0)
    def _init():
        ssum_ref[...] = jnp.zeros_like(ssum_ref)
        ssq_ref[...] = jnp.zeros_like(ssq_ref)

    ssum_ref[...] += (M * jnp.sum(base, axis=0, keepdims=True)
                      + jnp.sum(s1, axis=0, keepdims=True))
    ssq_ref[...] += (M * jnp.sum(base * base, axis=0, keepdims=True)
                     + 2.0 * jnp.sum(base * s1, axis=0, keepdims=True)
                     + jnp.sum(x * x, axis=0, keepdims=True))


def _main_body(base_ref, g_ref, nbrT_ref, idxf_ref, idx_ref, wn_ref,
               we_ref, a_ref, bb_ref, s_ref, tsum_ref, tsq_ref):
    pid = pl.program_id(0)
    maskT = (idxf_ref[...] != 0).astype(jnp.bfloat16)
    nbrT = nbrT_ref[...].astype(jnp.bfloat16) * maskT
    gbf = g_ref[...].astype(jnp.bfloat16)
    x = jnp.dot(gbf, wn_ref[...], preferred_element_type=jnp.float32)
    x = x + lax.dot_general(nbrT, we_ref[...],
                            (((0,), (0,)), ((), ())),
                            preferred_element_type=jnp.float32)
    yb = base_ref[...] * a_ref[...] + bb_ref[...]
    rep = jnp.broadcast_to(yb[:, None, :], (B, M, F)).reshape(BM, F)
    y = rep + x

    p = jax.nn.sigmoid(y[:, :D]) * jax.nn.softplus(y[:, D:])
    psum = jnp.sum(p.reshape(B, M, D), axis=1)
    cnt0 = jnp.sum((idx_ref[...] == 0).astype(jnp.float32), axis=1,
                   keepdims=True)
    corr = jax.nn.sigmoid(yb[:, :D]) * jax.nn.softplus(yb[:, D:])
    s = psum - cnt0 * corr
    s_ref[...] = s

    @pl.when(pid == 0)
    def _init():
        tsum_ref[...] = jnp.zeros_like(tsum_ref)
        tsq_ref[...] = jnp.zeros_like(tsq_ref)

    tsum_ref[...] += jnp.sum(s, axis=0, keepdims=True)
    tsq_ref[...] += jnp.sum(s * s, axis=0, keepdims=True)


def _final_body(atom_ref, s_ref, a2_ref, bb2_ref, out_ref):
    y2 = s_ref[...] * a2_ref[...] + bb2_ref[...]
    out_ref[...] = jax.nn.softplus(atom_ref[...] + y2)


def _stats_call(h, atom, g_h, nbrT, idxf, Ws, Wnb, We, bvec):
    oa = h * NBH
    return pl.pallas_call(
        _stats_body,
        grid=(NBH,),
        in_specs=[
            pl.BlockSpec((B, D), lambda b: (b + oa, 0)),
            pl.BlockSpec((BM, D), lambda b: (b, 0)),
            pl.BlockSpec((DE, BM), lambda b: (0, b + oa)),
            pl.BlockSpec((1, BM), lambda b: (0, b + oa)),
            pl.BlockSpec((D, F), lambda b: (0, 0)),
            pl.BlockSpec((D, F), lambda b: (0, 0)),
            pl.BlockSpec((DE, F), lambda b: (0, 0)),
            pl.BlockSpec((1, F), lambda b: (0, 0)),
        ],
        out_specs=[
            pl.BlockSpec((B, F), lambda b: (b, 0)),
            pl.BlockSpec((1, F), lambda b: (0, 0)),
            pl.BlockSpec((1, F), lambda b: (0, 0)),
        ],
        out_shape=[
            jax.ShapeDtypeStruct((NH, F), jnp.float32),
            jax.ShapeDtypeStruct((1, F), jnp.float32),
            jax.ShapeDtypeStruct((1, F), jnp.float32),
        ],
    )(atom, g_h, nbrT, idxf, Ws, Wnb, We, bvec)


def _main_call(h, base_h, g_h, nbrT, idxf, idx, Wn_s, We_s, a1, bb1):
    oa = h * NBH
    return pl.pallas_call(
        _main_body,
        grid=(NBH,),
        in_specs=[
            pl.BlockSpec((B, F), lambda b: (b, 0)),
            pl.BlockSpec((BM, D), lambda b: (b, 0)),
            pl.BlockSpec((DE, BM), lambda b: (0, b + oa)),
            pl.BlockSpec((1, BM), lambda b: (0, b + oa)),
            pl.BlockSpec((B, M), lambda b: (b + oa, 0)),
            pl.BlockSpec((D, F), lambda b: (0, 0)),
            pl.BlockSpec((DE, F), lambda b: (0, 0)),
            pl.BlockSpec((1, F), lambda b: (0, 0)),
            pl.BlockSpec((1, F), lambda b: (0, 0)),
        ],
        out_specs=[
            pl.BlockSpec((B, D), lambda b: (b, 0)),
            pl.BlockSpec((1, D), lambda b: (0, 0)),
            pl.BlockSpec((1, D), lambda b: (0, 0)),
        ],
        out_shape=[
            jax.ShapeDtypeStruct((NH, D), jnp.float32),
            jax.ShapeDtypeStruct((1, D), jnp.float32),
            jax.ShapeDtypeStruct((1, D), jnp.float32),
        ],
    )(base_h, g_h, nbrT, idxf, idx, Wn_s, We_s, a1, bb1)


def kernel(atom_in_fea, nbr_fea, nbr_fea_idx, W_fc, b_fc, gamma1, beta1,
           gamma2, beta2):
    idx = nbr_fea_idx.astype(jnp.int32)
    iflat = jnp.where(idx == 0, N, idx).reshape(HALVES, NCHUNKS_H, CHUNK)
    table = jnp.concatenate(
        [atom_in_fea, jnp.zeros((1, D), jnp.float32)], axis=0)
    nbrT = jnp.transpose(nbr_fea, (2, 0, 1)).reshape(DE, NM)
    idxf = idx.reshape(1, NM)

    Ws = W_fc[:, :D].T
    Wn = W_fc[:, D:2 * D].T
    Wnb = Wn.astype(jnp.bfloat16)
    We = W_fc[:, 2 * D:].T.astype(jnp.bfloat16)
    bvec = b_fc.reshape(1, F)

    gather = _make_sc_gather()
    g = [gather(table, iflat[h]) for h in range(HALVES)]

    stats = [
        _stats_call(h, atom_in_fea, g[h], nbrT, idxf, Ws, Wnb, We, bvec)
        for h in range(HALVES)
    ]
    ssum = sum(st[1] for st in stats)
    ssq = sum(st[2] for st in stats)

    mu1 = ssum / NM
    var1 = ssq / NM - mu1 * mu1
    a1 = lax.rsqrt(var1 + 1e-5) * gamma1.reshape(1, F)
    bb1 = beta1.reshape(1, F) - mu1 * a1
    Wn_s = (Wn * a1).astype(jnp.bfloat16)
    We_s = (We.astype(jnp.float32) * a1).astype(jnp.bfloat16)

    mains = [
        _main_call(h, stats[h][0], g[h], nbrT, idxf, idx, Wn_s, We_s, a1, bb1)
        for h in range(HALVES)
    ]
    tsum = sum(mn[1] for mn in mains)
    tsq = sum(mn[2] for mn in mains)

    mu2 = tsum / N
    var2 = tsq / N - mu2 * mu2
    a2 = lax.rsqrt(var2 + 1e-5) * gamma2.reshape(1, D)
    bb2 = beta2.reshape(1, D) - mu2 * a2

    s_all = jnp.concatenate([mn[0] for mn in mains], axis=0)
    out = pl.pallas_call(
        _final_body,
        grid=(HALVES * NBH,),
        in_specs=[
            pl.BlockSpec((B, D), lambda b: (b, 0)),
            pl.BlockSpec((B, D), lambda b: (b, 0)),
            pl.BlockSpec((1, D), lambda b: (0, 0)),
            pl.BlockSpec((1, D), lambda b: (0, 0)),
        ],
        out_specs=pl.BlockSpec((B, D), lambda b: (b, 0)),
        out_shape=jax.ShapeDtypeStruct((N, D), jnp.float32),
    )(atom_in_fea, s_all, a2, bb2)
    return out
